# baseline (speedup 1.0000x reference)
import math

import jax
import jax.numpy as jnp
from jax import lax
from jax.experimental import pallas as pl
from jax.experimental.pallas import tpu as pltpu

_ROWS = 1000000
_COLS = 64
_B = 5000
_K = _ROWS // _B
_NBUF = 4


def _make_body(cpc):
    def body(x_hbm, y_hbm, cache_hbm, out_hbm,
             xy_v, in_buf, out_buf, in_sems, out_sems):
        core = lax.axis_index("core")
        first = core * cpc

        def in_copy(j, slot):
            g = first + j
            return pltpu.make_async_copy(
                cache_hbm.at[pl.ds(g * _B, _B), :], in_buf.at[slot],
                in_sems.at[slot])

        def out_copy(j, slot):
            g = first + j
            return pltpu.make_async_copy(
                out_buf.at[slot], out_hbm.at[pl.ds(g * _B, _B), :],
                out_sems.at[slot])

        @pl.when(core == 0)
        def _():
            pltpu.sync_copy(x_hbm, xy_v.at[0])
            pltpu.sync_copy(y_hbm, xy_v.at[1])

        for s in range(_NBUF):
            in_copy(s, s).start()

        def step(j, carry):
            slot = lax.rem(j, _NBUF)
            in_copy(j, slot).wait()

            @pl.when(j >= _NBUF)
            def _():
                out_copy(j - _NBUF, slot).wait()

            out_buf[slot] = jnp.maximum(in_buf[slot], 0.0)

            @pl.when(jnp.logical_and(core == 0, j == 0))
            def _():
                m = xy_v[0] * xy_v[1]
                out_buf[0, 0:1, :] = jnp.maximum(m, 0.0)

            out_copy(j, slot).start()

            @pl.when(j + _NBUF < cpc)
            def _():
                in_copy(j + _NBUF, slot).start()

            return carry

        lax.fori_loop(0, cpc, step, 0)

        for s in range(_NBUF):
            j = cpc - _NBUF + s
            out_copy(j, j % _NBUF).wait()

    return body


def kernel(x, y, cache):
    mesh = pltpu.create_tensorcore_mesh("core")
    nc = math.prod(mesh.shape.values())
    cpc = _K // nc
    assert _K % nc == 0 and cpc >= _NBUF
    return pl.kernel(
        _make_body(cpc),
        out_type=jax.ShapeDtypeStruct((_ROWS, _COLS), jnp.float32),
        mesh=mesh,
        scratch_types=[
            pltpu.VMEM((2, 1, _COLS), jnp.float32),
            pltpu.VMEM((_NBUF, _B, _COLS), jnp.float32),
            pltpu.VMEM((_NBUF, _B, _COLS), jnp.float32),
            pltpu.SemaphoreType.DMA((_NBUF,)),
            pltpu.SemaphoreType.DMA((_NBUF,)),
        ],
    )(x, y, cache)
